# baseline (speedup 1.0000x reference)
import functools

import jax
import jax.numpy as jnp
from jax import lax
from jax.experimental import pallas as pl
from jax.experimental.pallas import tpu as pltpu
from jax.experimental.pallas import tpu_sc as plsc

NC = 2
NS = 16
NW = NC * NS
L = 16

GW = 128
KW = 2
HROWS = 5120
ACC_ROWS = HROWS + 128


def _sc_mesh():
    return plsc.VectorSubcoreMesh(
        core_axis_name="c", subcore_axis_name="s", num_cores=NC
    )


KG = 2


def _sc_gather(table, gidx2d):
    nwin, _ = gidx2d.shape
    B = nwin * GW
    D = table.shape[1]
    ngrp = nwin // KG
    niter = (ngrp + NW - 1) // NW
    nout = (niter + 1) // 2

    @functools.partial(
        pl.kernel,
        mesh=_sc_mesh(),
        out_type=jax.ShapeDtypeStruct((B, D), table.dtype),
        scratch_types=[
            pltpu.VMEM((2, KG, GW), jnp.int32),
            pltpu.VMEM((2, KG * GW, D), table.dtype),
            pltpu.SemaphoreType.DMA,
            pltpu.SemaphoreType.DMA,
            pltpu.SemaphoreType.DMA,
            pltpu.SemaphoreType.DMA,
            pltpu.SemaphoreType.DMA,
            pltpu.SemaphoreType.DMA,
        ],
    )
    def k(table_hbm, idx_hbm, out_hbm, idx_v, rows_v,
          si0, si1, sg0, sg1, so0, so1):
        si = (si0, si1)
        sg = (sg0, sg1)
        so = (so0, so1)
        wid = lax.axis_index("s") * NC + lax.axis_index("c")

        def idx_copy(g, b):
            return pltpu.make_async_copy(
                idx_hbm.at[pl.ds(g * KG, KG)], idx_v.at[b], si[b]
            )

        def out_copy(g, b):
            return pltpu.make_async_copy(
                rows_v.at[b], out_hbm.at[pl.ds(g * (KG * GW), KG * GW)], so[b]
            )

        def gat_copy(b, jj):
            return pltpu.make_async_copy(
                table_hbm.at[idx_v.at[b].at[jj]],
                rows_v.at[b].at[pl.ds(jj * GW, GW)],
                sg[b],
            )

        for b in range(2):
            g0 = wid + b * NW

            @pl.when(g0 < ngrp)
            def _(b=b, g0=g0):
                idx_copy(g0, b).start()

        @pl.loop(0, nout)
        def _(m):
            for b in range(2):
                i = 2 * m + b
                g = wid + i * NW

                @pl.when(g < ngrp)
                def _(b=b, i=i, g=g):
                    idx_copy(g, b).wait()

                    @pl.when(i >= 2)
                    def _():
                        out_copy(g - 2 * NW, b).wait()

                    for jj in range(KG):
                        gat_copy(b, jj).start()
                    for jj in range(KG):
                        gat_copy(b, jj).wait()
                    g2 = g + 2 * NW

                    @pl.when(g2 < ngrp)
                    def _():
                        idx_copy(g2, b).start()

                    out_copy(g, b).start()

        nt = jnp.maximum(0, (ngrp - wid + NW - 1) // NW)
        for b in range(2):
            @pl.when(nt >= b + 1)
            def _(b=b):
                out_copy(0, b).wait()

    return k(table, gidx2d)


def _sc_scatter_add(msg0, msg1, dst2d):
    E, D = msg0.shape
    nwin = dst2d.shape[0]
    ngrp = nwin // KW
    halfg = ngrp // 2
    niter = (ngrp + NS - 1) // NS
    nout = (niter + 1) // 2
    zrows = 128
    orows = HROWS // NS

    @functools.partial(
        pl.kernel,
        mesh=_sc_mesh(),
        out_type=jax.ShapeDtypeStruct((NC * HROWS, D), jnp.float32),
        scratch_types=[
            pltpu.VMEM((2, KW, GW), jnp.int32),
            pltpu.VMEM((KW, GW), jnp.int32),
            pltpu.VMEM((2, KW * GW, D), jnp.float32),
            pltpu.VMEM((zrows, D), jnp.float32),
            pltpu.VMEM_SHARED((ACC_ROWS, D), jnp.float32),
            pltpu.SemaphoreType.DMA,
            pltpu.SemaphoreType.DMA,
            pltpu.SemaphoreType.DMA,
            pltpu.SemaphoreType.DMA,
        ],
    )
    def k(m0_hbm, m1_hbm, idx_hbm, out_hbm, idx_v, lidx_v, msg_v, zbuf,
          shared, si0, si1, sm0, sm1):
        si = (si0, si1)
        sm = (sm0, sm1)
        c = lax.axis_index("c")
        s = lax.axis_index("s")
        vbase = jnp.full((L,), c * HROWS, jnp.int32)
        vabs = jnp.full((L,), HROWS + s, jnp.int32)
        vlim = jnp.full((L,), HROWS, jnp.int32)
        vzero = jnp.zeros((L,), jnp.int32)

        @pl.loop(0, zrows)
        def _(r):
            @pl.loop(0, D, step=L)
            def _(col):
                zbuf[r, pl.ds(col, L)] = jnp.zeros((L,), jnp.float32)

        rows_per_tile = ACC_ROWS // NS
        for b in range(rows_per_tile // zrows):
            pltpu.sync_copy(
                zbuf, shared.at[pl.ds(s * rows_per_tile + b * zrows, zrows)]
            )
        rem = rows_per_tile % zrows
        if rem:
            pltpu.sync_copy(
                zbuf.at[pl.ds(0, rem)],
                shared.at[
                    pl.ds(s * rows_per_tile + (rows_per_tile // zrows) * zrows,
                          rem)
                ],
            )
        plsc.subcore_barrier()

        def start_in(g, b):
            @pl.when(g < halfg)
            def _():
                pltpu.make_async_copy(
                    m0_hbm.at[pl.ds(g * (KW * GW), KW * GW)],
                    msg_v.at[b], sm[b],
                ).start()

            @pl.when(g >= halfg)
            def _():
                pltpu.make_async_copy(
                    m1_hbm.at[pl.ds(g * (KW * GW), KW * GW)],
                    msg_v.at[b], sm[b],
                ).start()

            pltpu.make_async_copy(
                idx_hbm.at[pl.ds(g * KW, KW)], idx_v.at[b], si[b]
            ).start()

        def wait_in(g, b):
            pltpu.make_async_copy(
                m0_hbm.at[pl.ds(0, KW * GW)], msg_v.at[b], sm[b]
            ).wait()
            pltpu.make_async_copy(
                idx_hbm.at[pl.ds(0, KW)], idx_v.at[b], si[b]
            ).wait()

        for b in range(2):
            g0 = s + b * NS

            @pl.when(g0 < ngrp)
            def _(b=b, g0=g0):
                start_in(g0, b)

        @pl.loop(0, nout)
        def _(m):
            for b in range(2):
                i = 2 * m + b
                g = s + i * NS

                @pl.when(g < ngrp)
                def _(b=b, g=g):
                    wait_in(g, b)
                    for jj in range(KW):
                        for kk in range(GW // L):
                            iv = idx_v[b, jj, pl.ds(kk * L, L)]
                            t = iv - vbase
                            valid = (t >= vzero) & (t < vlim)
                            lidx_v[jj, pl.ds(kk * L, L)] = jnp.where(
                                valid, t, vabs
                            )
                    for jj in range(KW):
                        pltpu.sync_copy(
                            msg_v.at[b].at[pl.ds(jj * GW, GW)],
                            shared.at[lidx_v.at[jj]],
                            add=True,
                        )
                    g2 = g + 2 * NS

                    @pl.when(g2 < ngrp)
                    def _():
                        start_in(g2, b)

        plsc.subcore_barrier()

        for b in range(orows // zrows):
            row = s * orows + b * zrows
            pltpu.sync_copy(shared.at[pl.ds(row, zrows)], zbuf)
            pltpu.sync_copy(zbuf, out_hbm.at[pl.ds(c * HROWS + row, zrows)])
        orem = orows % zrows
        if orem:
            row = s * orows + (orows // zrows) * zrows
            pltpu.sync_copy(shared.at[pl.ds(row, orem)], zbuf.at[pl.ds(0, orem)])
            pltpu.sync_copy(
                zbuf.at[pl.ds(0, orem)],
                out_hbm.at[pl.ds(c * HROWS + row, orem)],
            )

    return k(msg0, msg1, dst2d)


def _silu(x):
    return x * (1.0 / (1.0 + jnp.exp(-x)))


def _tables_body(h_ref, wa_ref, wb_ref, out_ref):
    h = h_ref[...]
    out_ref[0] = jnp.dot(h, wa_ref[...], preferred_element_type=jnp.float32)
    out_ref[1] = jnp.dot(h, wb_ref[...], preferred_element_type=jnp.float32)


def _msg_body(gs_ref, gd_ref, he_ref, w1c_ref, b1_ref, w2_ref, b2_ref, out_ref):
    pre = (
        gs_ref[0]
        + gd_ref[0]
        + jnp.dot(he_ref[...], w1c_ref[...], preferred_element_type=jnp.float32)
        + b1_ref[...]
    )
    h = _silu(pre)
    out_ref[...] = (
        jnp.dot(h, w2_ref[...], preferred_element_type=jnp.float32) + b2_ref[...]
    )


def _layernorm(z, g, b):
    m = jnp.mean(z, axis=-1, keepdims=True)
    zc = z - m
    v = jnp.mean(zc * zc, axis=-1, keepdims=True)
    return zc * lax.rsqrt(v + 1e-5) * g + b


def _node_body(h_ref, agg_ref, u1a_ref, u1b_ref, ub1_ref, u2_ref,
               ub2_ref, g_ref, b_ref, e1a_ref, e1b_ref, hn_ref, t2_ref):
    h = h_ref[...]
    agg = agg_ref[...]
    pre = (
        jnp.dot(h, u1a_ref[...], preferred_element_type=jnp.float32)
        + jnp.dot(agg, u1b_ref[...], preferred_element_type=jnp.float32)
        + ub1_ref[...]
    )
    hh = _silu(pre)
    y = jnp.dot(hh, u2_ref[...], preferred_element_type=jnp.float32) + ub2_ref[...]
    hn = _layernorm(h + y, g_ref[...], b_ref[...])
    hn_ref[...] = hn
    t2_ref[0] = jnp.dot(hn, e1a_ref[...], preferred_element_type=jnp.float32)
    t2_ref[1] = jnp.dot(hn, e1b_ref[...], preferred_element_type=jnp.float32)


def _edge_body(gs_ref, gd_ref, he_ref, w1c_ref, b1_ref, w2_ref, b2_ref,
               g_ref, bb_ref, alias_ref, out_ref):
    del alias_ref
    he = he_ref[...]
    pre = (
        gs_ref[0]
        + gd_ref[0]
        + jnp.dot(he, w1c_ref[...], preferred_element_type=jnp.float32)
        + b1_ref[...]
    )
    h = _silu(pre)
    y = jnp.dot(h, w2_ref[...], preferred_element_type=jnp.float32) + b2_ref[...]
    out_ref[...] = _layernorm(he + y, g_ref[...], bb_ref[...])


def _full(shape):
    return pl.BlockSpec(shape, lambda i: tuple(0 for _ in shape))


def kernel(h_node, h_edge, edge_index, msg_w1, msg_b1, msg_w2, msg_b2,
           upd_w1, upd_b1, upd_w2, upd_b2, eupd_w1, eupd_b1, eupd_w2,
           eupd_b2, nn_g, nn_b, en_g, en_b):
    N, D = h_node.shape
    E = h_edge.shape[0]
    Ec = E // 2

    src = edge_index[:, 0].astype(jnp.int32)
    dst = edge_index[:, 1].astype(jnp.int32)
    srcs = src.reshape(2, Ec)
    dsts = dst.reshape(2, Ec)
    gidx_c = [
        jnp.concatenate([srcs[c], dsts[c] + N]).reshape(2 * Ec // GW, GW)
        for c in (0, 1)
    ]
    dst2d = dst.reshape(E // GW, GW)

    w1a, w1b, w1c = msg_w1[:D], msg_w1[D:2 * D], msg_w1[2 * D:]
    u1a, u1b = upd_w1[:D], upd_w1[D:]
    e1a, e1b, e1c = eupd_w1[:D], eupd_w1[D:2 * D], eupd_w1[2 * D:]
    b1 = msg_b1.reshape(1, D)
    b2 = msg_b2.reshape(1, D)
    ub1 = upd_b1.reshape(1, D)
    ub2 = upd_b2.reshape(1, D)
    eb1 = eupd_b1.reshape(1, D)
    eb2 = eupd_b2.reshape(1, D)
    nng = nn_g.reshape(1, D)
    nnb = nn_b.reshape(1, D)
    eng = en_g.reshape(1, D)
    enb = en_b.reshape(1, D)

    BN = 2000
    BE = 2000
    gn = N // BN
    gec = Ec // BE

    t1 = pl.pallas_call(
        _tables_body,
        grid=(gn,),
        in_specs=[
            pl.BlockSpec((BN, D), lambda i: (i, 0)),
            _full((D, D)),
            _full((D, D)),
        ],
        out_specs=pl.BlockSpec((2, BN, D), lambda i: (0, i, 0)),
        out_shape=jax.ShapeDtypeStruct((2, N, D), jnp.float32),
    )(h_node, w1a, w1b)
    t1 = t1.reshape(2 * N, D)

    g1c = [_sc_gather(t1, gidx_c[c]).reshape(2, Ec, D) for c in (0, 1)]

    def _msg_call(g1x, c):
        return pl.pallas_call(
            _msg_body,
            grid=(gec,),
            in_specs=[
                pl.BlockSpec((1, BE, D), lambda i: (0, i, 0)),
                pl.BlockSpec((1, BE, D), lambda i: (1, i, 0)),
                pl.BlockSpec((BE, D), lambda i, c=c: (c * gec + i, 0)),
                _full((D, D)),
                _full((1, D)),
                _full((D, D)),
                _full((1, D)),
            ],
            out_specs=pl.BlockSpec((BE, D), lambda i, c=c: (c * gec + i, 0)),
            out_shape=jax.ShapeDtypeStruct((E, D), jnp.float32),
        )(g1x, g1x, h_edge, w1c, b1, msg_w2, b2)

    msg0 = _msg_call(g1c[0], 0)
    msg1 = _msg_call(g1c[1], 1)

    agg = _sc_scatter_add(msg0, msg1, dst2d)[:N]

    h_node_new, t2 = pl.pallas_call(
        _node_body,
        grid=(gn,),
        in_specs=[
            pl.BlockSpec((BN, D), lambda i: (i, 0)),
            pl.BlockSpec((BN, D), lambda i: (i, 0)),
            _full((D, D)),
            _full((D, D)),
            _full((1, D)),
            _full((D, D)),
            _full((1, D)),
            _full((1, D)),
            _full((1, D)),
            _full((D, D)),
            _full((D, D)),
        ],
        out_specs=[
            pl.BlockSpec((BN, D), lambda i: (i, 0)),
            pl.BlockSpec((2, BN, D), lambda i: (0, i, 0)),
        ],
        out_shape=[
            jax.ShapeDtypeStruct((N, D), jnp.float32),
            jax.ShapeDtypeStruct((2, N, D), jnp.float32),
        ],
    )(h_node, agg, u1a, u1b, ub1, upd_w2, ub2, nng, nnb, e1a, e1b)
    t2 = t2.reshape(2 * N, D)

    g2c = [_sc_gather(t2, gidx_c[c]).reshape(2, Ec, D) for c in (0, 1)]

    def _edge_call(g2x, c, alias_buf):
        return pl.pallas_call(
            _edge_body,
            grid=(gec,),
            in_specs=[
                pl.BlockSpec((1, BE, D), lambda i: (0, i, 0)),
                pl.BlockSpec((1, BE, D), lambda i: (1, i, 0)),
                pl.BlockSpec((BE, D), lambda i, c=c: (c * gec + i, 0)),
                _full((D, D)),
                _full((1, D)),
                _full((D, D)),
                _full((1, D)),
                _full((1, D)),
                _full((1, D)),
                pl.BlockSpec(memory_space=pl.ANY),
            ],
            out_specs=pl.BlockSpec((BE, D), lambda i, c=c: (c * gec + i, 0)),
            out_shape=jax.ShapeDtypeStruct((E, D), jnp.float32),
            input_output_aliases={9: 0},
        )(g2x, g2x, h_edge, e1c, eb1, eupd_w2, eb2, eng, enb, alias_buf)

    e0 = _edge_call(g2c[0], 0, msg0)
    h_edge_new = _edge_call(g2c[1], 1, e0)

    return (h_node_new, h_edge_new)


# speedup vs baseline: 1.0393x; 1.0393x over previous
import functools

import jax
import jax.numpy as jnp
from jax import lax
from jax.experimental import pallas as pl
from jax.experimental.pallas import tpu as pltpu
from jax.experimental.pallas import tpu_sc as plsc

NC = 2
NS = 16
NW = NC * NS
L = 16

GW = 128
KW = 2
HROWS = 5120
ACC_ROWS = HROWS + 128


def _sc_mesh():
    return plsc.VectorSubcoreMesh(
        core_axis_name="c", subcore_axis_name="s", num_cores=NC
    )


KG = 2


def _sc_gather(table, gidx2d):
    nwin, _ = gidx2d.shape
    B = nwin * GW
    D = table.shape[1]
    ngrp = nwin // KG
    niter = (ngrp + NW - 1) // NW
    nout = (niter + 1) // 2

    @functools.partial(
        pl.kernel,
        mesh=_sc_mesh(),
        out_type=jax.ShapeDtypeStruct((B, D), table.dtype),
        scratch_types=[
            pltpu.VMEM((2, KG, GW), jnp.int32),
            pltpu.VMEM((2, KG * GW, D), table.dtype),
            pltpu.SemaphoreType.DMA,
            pltpu.SemaphoreType.DMA,
            pltpu.SemaphoreType.DMA,
            pltpu.SemaphoreType.DMA,
            pltpu.SemaphoreType.DMA,
            pltpu.SemaphoreType.DMA,
        ],
    )
    def k(table_hbm, idx_hbm, out_hbm, idx_v, rows_v,
          si0, si1, sg0, sg1, so0, so1):
        si = (si0, si1)
        sg = (sg0, sg1)
        so = (so0, so1)
        wid = lax.axis_index("s") * NC + lax.axis_index("c")

        def idx_copy(g, b):
            return pltpu.make_async_copy(
                idx_hbm.at[pl.ds(g * KG, KG)], idx_v.at[b], si[b]
            )

        def out_copy(g, b):
            return pltpu.make_async_copy(
                rows_v.at[b], out_hbm.at[pl.ds(g * (KG * GW), KG * GW)], so[b]
            )

        def gat_copy(b, jj):
            return pltpu.make_async_copy(
                table_hbm.at[idx_v.at[b].at[jj]],
                rows_v.at[b].at[pl.ds(jj * GW, GW)],
                sg[b],
            )

        for b in range(2):
            g0 = wid + b * NW

            @pl.when(g0 < ngrp)
            def _(b=b, g0=g0):
                idx_copy(g0, b).start()

        @pl.loop(0, nout)
        def _(m):
            for b in range(2):
                i = 2 * m + b
                g = wid + i * NW

                @pl.when(g < ngrp)
                def _(b=b, i=i, g=g):
                    idx_copy(g, b).wait()

                    @pl.when(i >= 2)
                    def _():
                        out_copy(g - 2 * NW, b).wait()

                    for jj in range(KG):
                        gat_copy(b, jj).start()
                    for jj in range(KG):
                        gat_copy(b, jj).wait()
                    g2 = g + 2 * NW

                    @pl.when(g2 < ngrp)
                    def _():
                        idx_copy(g2, b).start()

                    out_copy(g, b).start()

        nt = jnp.maximum(0, (ngrp - wid + NW - 1) // NW)
        for b in range(2):
            @pl.when(nt >= b + 1)
            def _(b=b):
                out_copy(0, b).wait()

    return k(table, gidx2d)


def _sc_scatter_add(msg, dst2d, row0):
    _, D = msg.shape
    nwin = dst2d.shape[0]
    ngrp = nwin // KW
    niter = (ngrp + NS - 1) // NS
    nout = (niter + 1) // 2
    zrows = 128
    orows = HROWS // NS

    @functools.partial(
        pl.kernel,
        mesh=_sc_mesh(),
        out_type=jax.ShapeDtypeStruct((NC * HROWS, D), jnp.float32),
        scratch_types=[
            pltpu.VMEM((2, KW, GW), jnp.int32),
            pltpu.VMEM((KW, GW), jnp.int32),
            pltpu.VMEM((2, KW * GW, D), jnp.float32),
            pltpu.VMEM((zrows, D), jnp.float32),
            pltpu.VMEM_SHARED((ACC_ROWS, D), jnp.float32),
            pltpu.SemaphoreType.DMA,
            pltpu.SemaphoreType.DMA,
            pltpu.SemaphoreType.DMA,
            pltpu.SemaphoreType.DMA,
        ],
    )
    def k(msg_hbm, idx_hbm, out_hbm, idx_v, lidx_v, msg_v, zbuf, shared,
          si0, si1, sm0, sm1):
        si = (si0, si1)
        sm = (sm0, sm1)
        c = lax.axis_index("c")
        s = lax.axis_index("s")
        vbase = jnp.full((L,), c * HROWS, jnp.int32)
        vabs = jnp.full((L,), HROWS + s, jnp.int32)
        vlim = jnp.full((L,), HROWS, jnp.int32)
        vzero = jnp.zeros((L,), jnp.int32)

        @pl.loop(0, zrows)
        def _(r):
            @pl.loop(0, D, step=L)
            def _(col):
                zbuf[r, pl.ds(col, L)] = jnp.zeros((L,), jnp.float32)

        rows_per_tile = ACC_ROWS // NS
        for b in range(rows_per_tile // zrows):
            pltpu.sync_copy(
                zbuf, shared.at[pl.ds(s * rows_per_tile + b * zrows, zrows)]
            )
        rem = rows_per_tile % zrows
        if rem:
            pltpu.sync_copy(
                zbuf.at[pl.ds(0, rem)],
                shared.at[
                    pl.ds(s * rows_per_tile + (rows_per_tile // zrows) * zrows,
                          rem)
                ],
            )
        plsc.subcore_barrier()

        def in_copies(g, b):
            return (
                pltpu.make_async_copy(
                    msg_hbm.at[pl.ds(row0 + g * (KW * GW), KW * GW)],
                    msg_v.at[b], sm[b],
                ),
                pltpu.make_async_copy(
                    idx_hbm.at[pl.ds(g * KW, KW)], idx_v.at[b], si[b]
                ),
            )

        for b in range(2):
            g0 = s + b * NS

            @pl.when(g0 < ngrp)
            def _(b=b, g0=g0):
                for cp in in_copies(g0, b):
                    cp.start()

        @pl.loop(0, nout)
        def _(m):
            for b in range(2):
                i = 2 * m + b
                g = s + i * NS

                @pl.when(g < ngrp)
                def _(b=b, g=g):
                    for cp in in_copies(g, b):
                        cp.wait()
                    for jj in range(KW):
                        for kk in range(GW // L):
                            iv = idx_v[b, jj, pl.ds(kk * L, L)]
                            t = iv - vbase
                            valid = (t >= vzero) & (t < vlim)
                            lidx_v[jj, pl.ds(kk * L, L)] = jnp.where(
                                valid, t, vabs
                            )
                    for jj in range(KW):
                        pltpu.sync_copy(
                            msg_v.at[b].at[pl.ds(jj * GW, GW)],
                            shared.at[lidx_v.at[jj]],
                            add=True,
                        )
                    g2 = g + 2 * NS

                    @pl.when(g2 < ngrp)
                    def _():
                        for cp in in_copies(g2, b):
                            cp.start()

        plsc.subcore_barrier()

        for b in range(orows // zrows):
            row = s * orows + b * zrows
            pltpu.sync_copy(shared.at[pl.ds(row, zrows)], zbuf)
            pltpu.sync_copy(zbuf, out_hbm.at[pl.ds(c * HROWS + row, zrows)])
        orem = orows % zrows
        if orem:
            row = s * orows + (orows // zrows) * zrows
            pltpu.sync_copy(shared.at[pl.ds(row, orem)], zbuf.at[pl.ds(0, orem)])
            pltpu.sync_copy(
                zbuf.at[pl.ds(0, orem)],
                out_hbm.at[pl.ds(c * HROWS + row, orem)],
            )

    return k(msg, dst2d)


def _silu(x):
    return x * (1.0 / (1.0 + jnp.exp(-x)))


def _tables_body(h_ref, wa_ref, wb_ref, out_ref):
    h = h_ref[...]
    out_ref[0] = jnp.dot(h, wa_ref[...], preferred_element_type=jnp.float32)
    out_ref[1] = jnp.dot(h, wb_ref[...], preferred_element_type=jnp.float32)


def _msg_body(gs_ref, gd_ref, he_ref, w1c_ref, b1_ref, w2_ref, b2_ref, out_ref):
    pre = (
        gs_ref[0]
        + gd_ref[0]
        + jnp.dot(he_ref[...], w1c_ref[...], preferred_element_type=jnp.float32)
        + b1_ref[...]
    )
    h = _silu(pre)
    out_ref[...] = (
        jnp.dot(h, w2_ref[...], preferred_element_type=jnp.float32) + b2_ref[...]
    )


def _layernorm(z, g, b):
    m = jnp.mean(z, axis=-1, keepdims=True)
    zc = z - m
    v = jnp.mean(zc * zc, axis=-1, keepdims=True)
    return zc * lax.rsqrt(v + 1e-5) * g + b


def _node_body(h_ref, p0_ref, p1_ref, u1a_ref, u1b_ref, ub1_ref, u2_ref,
               ub2_ref, g_ref, b_ref, e1a_ref, e1b_ref, hn_ref, t2_ref):
    h = h_ref[...]
    agg = p0_ref[...] + p1_ref[...]
    pre = (
        jnp.dot(h, u1a_ref[...], preferred_element_type=jnp.float32)
        + jnp.dot(agg, u1b_ref[...], preferred_element_type=jnp.float32)
        + ub1_ref[...]
    )
    hh = _silu(pre)
    y = jnp.dot(hh, u2_ref[...], preferred_element_type=jnp.float32) + ub2_ref[...]
    hn = _layernorm(h + y, g_ref[...], b_ref[...])
    hn_ref[...] = hn
    t2_ref[0] = jnp.dot(hn, e1a_ref[...], preferred_element_type=jnp.float32)
    t2_ref[1] = jnp.dot(hn, e1b_ref[...], preferred_element_type=jnp.float32)


def _edge_body(gs_ref, gd_ref, he_ref, w1c_ref, b1_ref, w2_ref, b2_ref,
               g_ref, bb_ref, alias_ref, out_ref):
    del alias_ref
    he = he_ref[...]
    pre = (
        gs_ref[0]
        + gd_ref[0]
        + jnp.dot(he, w1c_ref[...], preferred_element_type=jnp.float32)
        + b1_ref[...]
    )
    h = _silu(pre)
    y = jnp.dot(h, w2_ref[...], preferred_element_type=jnp.float32) + b2_ref[...]
    out_ref[...] = _layernorm(he + y, g_ref[...], bb_ref[...])


def _full(shape):
    return pl.BlockSpec(shape, lambda i: tuple(0 for _ in shape))


def kernel(h_node, h_edge, edge_index, msg_w1, msg_b1, msg_w2, msg_b2,
           upd_w1, upd_b1, upd_w2, upd_b2, eupd_w1, eupd_b1, eupd_w2,
           eupd_b2, nn_g, nn_b, en_g, en_b):
    N, D = h_node.shape
    E = h_edge.shape[0]
    Ec = E // 2

    src = edge_index[:, 0].astype(jnp.int32)
    dst = edge_index[:, 1].astype(jnp.int32)
    srcs = src.reshape(2, Ec)
    dsts = dst.reshape(2, Ec)
    gidx_c = [
        jnp.concatenate([srcs[c], dsts[c] + N]).reshape(2 * Ec // GW, GW)
        for c in (0, 1)
    ]
    dst2d_c = [dsts[c].reshape(Ec // GW, GW) for c in (0, 1)]

    w1a, w1b, w1c = msg_w1[:D], msg_w1[D:2 * D], msg_w1[2 * D:]
    u1a, u1b = upd_w1[:D], upd_w1[D:]
    e1a, e1b, e1c = eupd_w1[:D], eupd_w1[D:2 * D], eupd_w1[2 * D:]
    b1 = msg_b1.reshape(1, D)
    b2 = msg_b2.reshape(1, D)
    ub1 = upd_b1.reshape(1, D)
    ub2 = upd_b2.reshape(1, D)
    eb1 = eupd_b1.reshape(1, D)
    eb2 = eupd_b2.reshape(1, D)
    nng = nn_g.reshape(1, D)
    nnb = nn_b.reshape(1, D)
    eng = en_g.reshape(1, D)
    enb = en_b.reshape(1, D)

    BN = 2000
    BE = 2000
    gn = N // BN
    gec = Ec // BE

    t1 = pl.pallas_call(
        _tables_body,
        grid=(gn,),
        in_specs=[
            pl.BlockSpec((BN, D), lambda i: (i, 0)),
            _full((D, D)),
            _full((D, D)),
        ],
        out_specs=pl.BlockSpec((2, BN, D), lambda i: (0, i, 0)),
        out_shape=jax.ShapeDtypeStruct((2, N, D), jnp.float32),
    )(h_node, w1a, w1b)
    t1 = t1.reshape(2 * N, D)

    g1c = [_sc_gather(t1, gidx_c[c]).reshape(2, Ec, D) for c in (0, 1)]

    def _msg_call(g1x, c):
        return pl.pallas_call(
            _msg_body,
            grid=(gec,),
            in_specs=[
                pl.BlockSpec((1, BE, D), lambda i: (0, i, 0)),
                pl.BlockSpec((1, BE, D), lambda i: (1, i, 0)),
                pl.BlockSpec((BE, D), lambda i, c=c: (c * gec + i, 0)),
                _full((D, D)),
                _full((1, D)),
                _full((D, D)),
                _full((1, D)),
            ],
            out_specs=pl.BlockSpec((BE, D), lambda i, c=c: (c * gec + i, 0)),
            out_shape=jax.ShapeDtypeStruct((E, D), jnp.float32),
        )(g1x, g1x, h_edge, w1c, b1, msg_w2, b2)

    msg0 = _msg_call(g1c[0], 0)
    msg1 = _msg_call(g1c[1], 1)

    p0 = _sc_scatter_add(msg0, dst2d_c[0], 0)
    p1 = _sc_scatter_add(msg1, dst2d_c[1], Ec)

    h_node_new, t2 = pl.pallas_call(
        _node_body,
        grid=(gn,),
        in_specs=[
            pl.BlockSpec((BN, D), lambda i: (i, 0)),
            pl.BlockSpec((BN, D), lambda i: (i, 0)),
            pl.BlockSpec((BN, D), lambda i: (i, 0)),
            _full((D, D)),
            _full((D, D)),
            _full((1, D)),
            _full((D, D)),
            _full((1, D)),
            _full((1, D)),
            _full((1, D)),
            _full((D, D)),
            _full((D, D)),
        ],
        out_specs=[
            pl.BlockSpec((BN, D), lambda i: (i, 0)),
            pl.BlockSpec((2, BN, D), lambda i: (0, i, 0)),
        ],
        out_shape=[
            jax.ShapeDtypeStruct((N, D), jnp.float32),
            jax.ShapeDtypeStruct((2, N, D), jnp.float32),
        ],
    )(h_node, p0[:N], p1[:N], u1a, u1b, ub1, upd_w2, ub2, nng, nnb, e1a, e1b)
    t2 = t2.reshape(2 * N, D)

    g2c = [_sc_gather(t2, gidx_c[c]).reshape(2, Ec, D) for c in (0, 1)]

    def _edge_call(g2x, c, alias_buf):
        return pl.pallas_call(
            _edge_body,
            grid=(gec,),
            in_specs=[
                pl.BlockSpec((1, BE, D), lambda i: (0, i, 0)),
                pl.BlockSpec((1, BE, D), lambda i: (1, i, 0)),
                pl.BlockSpec((BE, D), lambda i, c=c: (c * gec + i, 0)),
                _full((D, D)),
                _full((1, D)),
                _full((D, D)),
                _full((1, D)),
                _full((1, D)),
                _full((1, D)),
                pl.BlockSpec(memory_space=pl.ANY),
            ],
            out_specs=pl.BlockSpec((BE, D), lambda i, c=c: (c * gec + i, 0)),
            out_shape=jax.ShapeDtypeStruct((E, D), jnp.float32),
            input_output_aliases={9: 0},
        )(g2x, g2x, h_edge, e1c, eb1, eupd_w2, eb2, eng, enb, alias_buf)

    e0 = _edge_call(g2c[0], 0, msg0)
    h_edge_new = _edge_call(g2c[1], 1, e0)

    return (h_node_new, h_edge_new)
